# baseline (speedup 1.0000x reference)
import functools

import jax
import jax.numpy as jnp
from jax import lax
from jax.experimental import pallas as pl
from jax.experimental.pallas import tpu as pltpu
from jax.experimental.pallas import tpu_sc as plsc

_FIELD_DIM = 38462
_NUM_FIELDS = 26
_BATCH = 16384
_EMBED = 16
_B = _BATCH * _NUM_FIELDS

_TC_COLS = 128
_TC_ROWS = _B // _TC_COLS
_TC_BLK = 256

def _idx_body(x_ref, o_ref):
    i = pl.program_id(0)
    rows = lax.broadcasted_iota(jnp.int32, (_TC_BLK, _TC_COLS), 0)
    lanes = lax.broadcasted_iota(jnp.int32, (_TC_BLK, _TC_COLS), 1)
    p = (i * _TC_BLK + rows) * _TC_COLS + lanes
    o_ref[...] = x_ref[...] + (p % _NUM_FIELDS) * _FIELD_DIM


def _compute_idx(x_flat2d):
    return pl.pallas_call(
        _idx_body,
        grid=(_TC_ROWS // _TC_BLK,),
        in_specs=[pl.BlockSpec((_TC_BLK, _TC_COLS), lambda i: (i, 0))],
        out_specs=pl.BlockSpec((_TC_BLK, _TC_COLS), lambda i: (i, 0)),
        out_shape=jax.ShapeDtypeStruct((_TC_ROWS, _TC_COLS), jnp.int32),
    )(x_flat2d)


_NW = 32
_BPW = _B // _NW
_CHUNK = 1664
_NCHUNK = _BPW // _CHUNK


@functools.partial(
    pl.kernel,
    mesh=plsc.VectorSubcoreMesh(core_axis_name="c", subcore_axis_name="s"),
    out_type=jax.ShapeDtypeStruct((_B, _EMBED), jnp.float32),
    scratch_types=[
        pltpu.VMEM((_CHUNK,), jnp.int32),
        pltpu.VMEM((_CHUNK, _EMBED), jnp.float32),
        pltpu.SemaphoreType.DMA,
    ],
    compiler_params=pltpu.CompilerParams(use_tc_tiling_on_sc=False),
)
def _sc_gather(idx_hbm, table_hbm, out_hbm, idx_v, rows_v, sem):
    wid = lax.axis_index("s") * 2 + lax.axis_index("c")
    base = wid * _BPW

    def chunk(g, carry):
        off = base + g * _CHUNK
        pltpu.sync_copy(idx_hbm.at[pl.ds(off, _CHUNK)], idx_v)
        pltpu.async_copy(table_hbm.at[idx_v], rows_v, sem).wait()
        pltpu.sync_copy(rows_v, out_hbm.at[pl.ds(off, _CHUNK)])
        return carry

    lax.fori_loop(0, _NCHUNK, chunk, 0)


def kernel(x, table):
    x_flat2d = x.reshape(_TC_ROWS, _TC_COLS)
    idx = _compute_idx(x_flat2d).reshape(_B)
    out_flat = _sc_gather(idx, table)
    return out_flat.reshape(_BATCH, _NUM_FIELDS, _EMBED)


# speedup vs baseline: 1.4421x; 1.4421x over previous
import functools

import jax
import jax.numpy as jnp
from jax import lax
from jax.experimental import pallas as pl
from jax.experimental.pallas import tpu as pltpu
from jax.experimental.pallas import tpu_sc as plsc

_FIELD_DIM = 38462
_NUM_FIELDS = 26
_BATCH = 16384
_EMBED = 16
_B = _BATCH * _NUM_FIELDS
_FPAD = 32

_TC_BLK = 2048

def _idx_body(xt_ref, o_ref):
    f = lax.broadcasted_iota(jnp.int32, (_NUM_FIELDS, _TC_BLK), 0)
    vals = xt_ref[...] + f * _FIELD_DIM
    pad = jnp.zeros((_FPAD - _NUM_FIELDS, _TC_BLK), jnp.int32)
    o_ref[...] = jnp.concatenate([vals, pad], axis=0)


def _compute_idx(xt):
    return pl.pallas_call(
        _idx_body,
        grid=(_BATCH // _TC_BLK,),
        in_specs=[pl.BlockSpec((_NUM_FIELDS, _TC_BLK), lambda i: (0, i))],
        out_specs=pl.BlockSpec((_FPAD, _TC_BLK), lambda i: (0, i)),
        out_shape=jax.ShapeDtypeStruct((_FPAD, _BATCH), jnp.int32),
    )(xt)


_NW = 32
_BPW = _B // _NW
_CHUNK = 1664
_NCHUNK = _BPW // _CHUNK


@functools.partial(
    pl.kernel,
    mesh=plsc.VectorSubcoreMesh(core_axis_name="c", subcore_axis_name="s"),
    out_type=jax.ShapeDtypeStruct((_B, _EMBED), jnp.float32),
    scratch_types=[
        pltpu.VMEM((_CHUNK,), jnp.int32),
        pltpu.VMEM((_CHUNK, _EMBED), jnp.float32),
        pltpu.SemaphoreType.DMA,
    ],
    compiler_params=pltpu.CompilerParams(use_tc_tiling_on_sc=False),
)
def _sc_gather(idx_hbm, table_hbm, out_hbm, idx_v, rows_v, sem):
    wid = lax.axis_index("s") * 2 + lax.axis_index("c")
    base = wid * _BPW

    def chunk(g, carry):
        off = base + g * _CHUNK
        pltpu.sync_copy(idx_hbm.at[pl.ds(off, _CHUNK)], idx_v)
        pltpu.async_copy(table_hbm.at[idx_v], rows_v, sem).wait()
        pltpu.sync_copy(rows_v, out_hbm.at[pl.ds(off, _CHUNK)])
        return carry

    lax.fori_loop(0, _NCHUNK, chunk, 0)


def kernel(x, table):
    idx = _compute_idx(x.T).reshape(_FPAD * _BATCH)
    out_flat = _sc_gather(idx, table)
    return out_flat.reshape(_NUM_FIELDS, _BATCH, _EMBED).transpose(1, 0, 2)


# speedup vs baseline: 1.6387x; 1.1363x over previous
import functools

import jax
import jax.numpy as jnp
from jax import lax
from jax.experimental import pallas as pl
from jax.experimental.pallas import tpu as pltpu
from jax.experimental.pallas import tpu_sc as plsc

_FIELD_DIM = 38462
_NUM_FIELDS = 26
_BATCH = 16384
_EMBED = 16
_B = _BATCH * _NUM_FIELDS
_FPAD = 32

_TC_BLK = 2048

def _idx_body(xt_ref, o_ref):
    f = lax.broadcasted_iota(jnp.int32, (_NUM_FIELDS, _TC_BLK), 0)
    vals = xt_ref[...] + f * _FIELD_DIM
    pad = jnp.zeros((_FPAD - _NUM_FIELDS, _TC_BLK), jnp.int32)
    o_ref[...] = jnp.concatenate([vals, pad], axis=0)


def _compute_idx(xt):
    return pl.pallas_call(
        _idx_body,
        grid=(_BATCH // _TC_BLK,),
        in_specs=[pl.BlockSpec((_NUM_FIELDS, _TC_BLK), lambda i: (0, i))],
        out_specs=pl.BlockSpec((_FPAD, _TC_BLK), lambda i: (0, i)),
        out_shape=jax.ShapeDtypeStruct((_FPAD, _BATCH), jnp.int32),
    )(xt)


_NW = 32
_BPW = _B // _NW
_CHUNK = 1664
_NBT = _CHUNK // 128
_NCHUNK = _BPW // _CHUNK
_NTILES = _NUM_FIELDS * 2 * (_BATCH // 128)


@functools.partial(
    pl.kernel,
    mesh=plsc.VectorSubcoreMesh(core_axis_name="c", subcore_axis_name="s"),
    out_type=jax.ShapeDtypeStruct((_NTILES, 8, 128), jnp.float32),
    scratch_types=[
        pltpu.VMEM((_CHUNK,), jnp.int32),
        pltpu.VMEM((_CHUNK, _EMBED), jnp.float32),
        pltpu.VMEM((_NBT, 2, 8, 128), jnp.float32),
        pltpu.SemaphoreType.DMA,
    ],
    compiler_params=pltpu.CompilerParams(
        use_tc_tiling_on_sc=False, needs_layout_passes=False
    ),
)
def _sc_gather(idx_hbm, table_hbm, out_hbm, idx_v, rows_v, tbuf, sem):
    wid = lax.axis_index("s") * 2 + lax.axis_index("c")
    base = wid * _BPW
    lane = lax.iota(jnp.int32, 16)
    cols = [jnp.full((16,), e, jnp.int32) for e in range(_EMBED)]

    def chunk(g, carry):
        off = base + g * _CHUNK
        pltpu.sync_copy(idx_hbm.at[pl.ds(off, _CHUNK)], idx_v)
        pltpu.async_copy(table_hbm.at[idx_v], rows_v, sem).wait()

        def btile(bt_local, c2):
            rbase = bt_local * 128
            for e in range(_EMBED):
                for j in range(8):
                    rid = rbase + j * 16 + lane
                    v = plsc.load_gather(rows_v, [rid, cols[e]])
                    tbuf[bt_local, e // 8, e % 8, pl.ds(j * 16, 16)] = v
            p_tile = off + rbase
            f = lax.shift_right_logical(p_tile, 14)
            bt = lax.shift_right_logical(p_tile & (_BATCH - 1), 7)
            t0 = f * 256 + bt
            pltpu.async_copy(tbuf.at[bt_local, 0], out_hbm.at[t0], sem)
            pltpu.async_copy(tbuf.at[bt_local, 1], out_hbm.at[t0 + 128], sem)
            return c2

        lax.fori_loop(0, _NBT, btile, 0)
        for _ in range(2 * _NBT):
            pltpu.make_async_copy(out_hbm.at[0], tbuf.at[0, 0], sem).wait()
        return carry

    lax.fori_loop(0, _NCHUNK, chunk, 0)


def kernel(x, table):
    idx = _compute_idx(x.T).reshape(_FPAD * _BATCH)
    out4 = _sc_gather(idx, table)
    out5 = out4.reshape(_NUM_FIELDS, 2, _BATCH // 128, 8, 128)
    return out5.transpose(2, 4, 0, 1, 3).reshape(_BATCH, _NUM_FIELDS, _EMBED)


# speedup vs baseline: 1.6874x; 1.0297x over previous
import functools

import jax
import jax.numpy as jnp
from jax import lax
from jax.experimental import pallas as pl
from jax.experimental.pallas import tpu as pltpu
from jax.experimental.pallas import tpu_sc as plsc

_FIELD_DIM = 38462
_NUM_FIELDS = 26
_BATCH = 16384
_EMBED = 16
_B = _BATCH * _NUM_FIELDS
_FPAD = 32

_TC_BLK = 2048

def _idx_body(xt_ref, o_ref):
    f = lax.broadcasted_iota(jnp.int32, (_NUM_FIELDS, _TC_BLK), 0)
    vals = xt_ref[...] + f * _FIELD_DIM
    pad = jnp.zeros((_FPAD - _NUM_FIELDS, _TC_BLK), jnp.int32)
    o_ref[...] = jnp.concatenate([vals, pad], axis=0)


def _compute_idx(xt):
    return pl.pallas_call(
        _idx_body,
        grid=(_BATCH // _TC_BLK,),
        in_specs=[pl.BlockSpec((_NUM_FIELDS, _TC_BLK), lambda i: (0, i))],
        out_specs=pl.BlockSpec((_FPAD, _TC_BLK), lambda i: (0, i)),
        out_shape=jax.ShapeDtypeStruct((_FPAD, _BATCH), jnp.int32),
    )(xt)


_NW = 32
_BPW = _B // _NW
_CHUNK = 1664
_NBT = _CHUNK // 128
_NCHUNK = _BPW // _CHUNK
_NTILES = _NUM_FIELDS * 2 * (_BATCH // 128)


@functools.partial(
    pl.kernel,
    mesh=plsc.VectorSubcoreMesh(core_axis_name="c", subcore_axis_name="s"),
    out_type=jax.ShapeDtypeStruct((_NTILES, 8, 128), jnp.float32),
    scratch_types=[
        pltpu.VMEM((2, _CHUNK), jnp.int32),
        pltpu.VMEM((2, _CHUNK, _EMBED), jnp.float32),
        pltpu.VMEM((2, _NBT, 2, 8, 128), jnp.float32),
        pltpu.SemaphoreType.DMA,
        pltpu.SemaphoreType.DMA,
        pltpu.SemaphoreType.DMA,
    ],
    compiler_params=pltpu.CompilerParams(
        use_tc_tiling_on_sc=False, needs_layout_passes=False
    ),
)
def _sc_gather(idx_hbm, table_hbm, out_hbm, idx_v, rows_v, tbuf, gsem, osem0, osem1):
    wid = lax.axis_index("s") * 2 + lax.axis_index("c")
    base = wid * _BPW
    lane = lax.iota(jnp.int32, 16)
    cols = [jnp.full((16,), e, jnp.int32) for e in range(_EMBED)]
    osems = [osem0, osem1]

    def transpose_and_fire(g, p):
        off = base + g * _CHUNK

        def btile(bt_local, c2):
            rbase = bt_local * 128
            for e in range(_EMBED):
                for j in range(8):
                    rid = rbase + j * 16 + lane
                    v = plsc.load_gather(rows_v.at[p], [rid, cols[e]])
                    tbuf[p, bt_local, e // 8, e % 8, pl.ds(j * 16, 16)] = v
            p_tile = off + rbase
            f = lax.shift_right_logical(p_tile, 14)
            bt = lax.shift_right_logical(p_tile & (_BATCH - 1), 7)
            t0 = f * 256 + bt
            pltpu.async_copy(tbuf.at[p, bt_local, 0], out_hbm.at[t0], osems[p])
            pltpu.async_copy(tbuf.at[p, bt_local, 1], out_hbm.at[t0 + 128], osems[p])
            return c2

        lax.fori_loop(0, _NBT, btile, 0)

    def drain_tiles(p):
        for _ in range(2 * _NBT):
            pltpu.make_async_copy(out_hbm.at[0], tbuf.at[0, 0, 0], osems[p]).wait()

    def wait_gather(p):
        pltpu.make_async_copy(
            table_hbm.at[idx_v.at[p]], rows_v.at[p], gsem
        ).wait()

    pltpu.sync_copy(idx_hbm.at[pl.ds(base, _CHUNK)], idx_v.at[0])
    pltpu.async_copy(table_hbm.at[idx_v.at[0]], rows_v.at[0], gsem)

    def super_step(k, carry):
        g0 = 2 * k
        g1 = g0 + 1
        pltpu.sync_copy(idx_hbm.at[pl.ds(base + g1 * _CHUNK, _CHUNK)], idx_v.at[1])
        wait_gather(0)
        pltpu.async_copy(table_hbm.at[idx_v.at[1]], rows_v.at[1], gsem)

        @pl.when(k >= 1)
        def _():
            drain_tiles(0)

        transpose_and_fire(g0, 0)

        @pl.when(k < _NCHUNK // 2 - 1)
        def _():
            pltpu.sync_copy(
                idx_hbm.at[pl.ds(base + (g1 + 1) * _CHUNK, _CHUNK)], idx_v.at[0]
            )
        wait_gather(1)

        @pl.when(k < _NCHUNK // 2 - 1)
        def _():
            pltpu.async_copy(table_hbm.at[idx_v.at[0]], rows_v.at[0], gsem)

        @pl.when(k >= 1)
        def _():
            drain_tiles(1)

        transpose_and_fire(g1, 1)
        return carry

    lax.fori_loop(0, _NCHUNK // 2, super_step, 0)
    drain_tiles(0)
    drain_tiles(1)


def kernel(x, table):
    idx = _compute_idx(x.T).reshape(_FPAD * _BATCH)
    out4 = _sc_gather(idx, table)
    out5 = out4.reshape(_NUM_FIELDS, 2, _BATCH // 128, 8, 128)
    return out5.transpose(2, 4, 0, 1, 3).reshape(_BATCH, _NUM_FIELDS, _EMBED)


# speedup vs baseline: 1.6881x; 1.0004x over previous
import functools

import jax
import jax.numpy as jnp
from jax import lax
from jax.experimental import pallas as pl
from jax.experimental.pallas import tpu as pltpu
from jax.experimental.pallas import tpu_sc as plsc

_FIELD_DIM = 38462
_NUM_FIELDS = 26
_BATCH = 16384
_EMBED = 16
_B = _BATCH * _NUM_FIELDS
_FPAD = 32

_TC_BLK = 2048

def _idx_body(xt_ref, o_ref):
    f = lax.broadcasted_iota(jnp.int32, (_NUM_FIELDS, _TC_BLK), 0)
    vals = xt_ref[...] + f * _FIELD_DIM
    pad = jnp.zeros((_FPAD - _NUM_FIELDS, _TC_BLK), jnp.int32)
    o_ref[...] = jnp.concatenate([vals, pad], axis=0)


def _compute_idx(xt):
    return pl.pallas_call(
        _idx_body,
        grid=(_BATCH // _TC_BLK,),
        in_specs=[pl.BlockSpec((_NUM_FIELDS, _TC_BLK), lambda i: (0, i))],
        out_specs=pl.BlockSpec((_FPAD, _TC_BLK), lambda i: (0, i)),
        out_shape=jax.ShapeDtypeStruct((_FPAD, _BATCH), jnp.int32),
    )(xt)


_NW = 32
_BPW = _B // _NW
_CHUNK = 1664
_NBT = _CHUNK // 128
_NCHUNK = _BPW // _CHUNK
_NTILES = _NUM_FIELDS * 2 * (_BATCH // 128)


@functools.partial(
    pl.kernel,
    mesh=plsc.VectorSubcoreMesh(core_axis_name="c", subcore_axis_name="s"),
    out_type=jax.ShapeDtypeStruct((_NTILES, 8, 128), jnp.float32),
    scratch_types=[
        pltpu.VMEM((2, _CHUNK), jnp.int32),
        pltpu.VMEM((2, _CHUNK, _EMBED), jnp.float32),
        pltpu.VMEM((2, _NBT, 2, 8, 128), jnp.float32),
        pltpu.SemaphoreType.DMA,
        pltpu.SemaphoreType.DMA,
        pltpu.SemaphoreType.DMA,
    ],
    compiler_params=pltpu.CompilerParams(
        use_tc_tiling_on_sc=False, needs_layout_passes=False
    ),
)
def _sc_gather(idx_hbm, table_hbm, out_hbm, idx_v, rows_v, tbuf, gsem, osem0, osem1):
    wid = lax.axis_index("s") * 2 + lax.axis_index("c")
    base = wid * _BPW
    lane = lax.iota(jnp.int32, 16)
    cols = [jnp.full((16,), e, jnp.int32) for e in range(_EMBED)]
    osems = [osem0, osem1]

    def transpose_and_fire(g, p):
        off = base + g * _CHUNK

        def btile(bt_local, c2):
            rbase = bt_local * 128
            rids = [rbase + j * 16 + lane for j in range(8)]
            for e in range(_EMBED):
                for j in range(8):
                    v = plsc.load_gather(rows_v.at[p], [rids[j], cols[e]])
                    tbuf[p, bt_local, e // 8, e % 8, pl.ds(j * 16, 16)] = v
            p_tile = off + rbase
            f = lax.shift_right_logical(p_tile, 14)
            bt = lax.shift_right_logical(p_tile & (_BATCH - 1), 7)
            t0 = f * 256 + bt
            pltpu.async_copy(tbuf.at[p, bt_local, 0], out_hbm.at[t0], osems[p])
            pltpu.async_copy(tbuf.at[p, bt_local, 1], out_hbm.at[t0 + 128], osems[p])
            return c2

        lax.fori_loop(0, _NBT, btile, 0)

    def drain_tiles(p):
        for _ in range(2 * _NBT):
            pltpu.make_async_copy(out_hbm.at[0], tbuf.at[0, 0, 0], osems[p]).wait()

    def wait_gather(p):
        pltpu.make_async_copy(
            table_hbm.at[idx_v.at[p]], rows_v.at[p], gsem
        ).wait()

    pltpu.sync_copy(idx_hbm.at[pl.ds(base, _CHUNK)], idx_v.at[0])
    pltpu.async_copy(table_hbm.at[idx_v.at[0]], rows_v.at[0], gsem)

    def super_step(k, carry):
        g0 = 2 * k
        g1 = g0 + 1
        pltpu.sync_copy(idx_hbm.at[pl.ds(base + g1 * _CHUNK, _CHUNK)], idx_v.at[1])
        wait_gather(0)
        pltpu.async_copy(table_hbm.at[idx_v.at[1]], rows_v.at[1], gsem)

        @pl.when(k >= 1)
        def _():
            drain_tiles(0)

        transpose_and_fire(g0, 0)

        @pl.when(k < _NCHUNK // 2 - 1)
        def _():
            pltpu.sync_copy(
                idx_hbm.at[pl.ds(base + (g1 + 1) * _CHUNK, _CHUNK)], idx_v.at[0]
            )
        wait_gather(1)

        @pl.when(k < _NCHUNK // 2 - 1)
        def _():
            pltpu.async_copy(table_hbm.at[idx_v.at[0]], rows_v.at[0], gsem)

        @pl.when(k >= 1)
        def _():
            drain_tiles(1)

        transpose_and_fire(g1, 1)
        return carry

    lax.fori_loop(0, _NCHUNK // 2, super_step, 0)
    drain_tiles(0)
    drain_tiles(1)


def kernel(x, table):
    idx = _compute_idx(x.T).reshape(_FPAD * _BATCH)
    out4 = _sc_gather(idx, table)
    out5 = out4.reshape(_NUM_FIELDS, 2, _BATCH // 128, 8, 128)
    return out5.transpose(2, 4, 0, 1, 3).reshape(_BATCH, _NUM_FIELDS, _EMBED)
